# baseline (speedup 1.0000x reference)
import functools

import jax
import jax.numpy as jnp
from jax import lax
from jax.experimental import pallas as pl
from jax.experimental.pallas import tpu as pltpu
from jax.experimental.pallas import tpu_sc as plsc

BATCH = 16384
EMBED = 32
NC, NS = 2, 16
NW = NC * NS
B_PER_W = BATCH // NW


def _sc_gather_body(uidx_hbm, iidx_hbm, utab_hbm, itab_hbm,
                    uout_hbm, iout_hbm,
                    uidx_v, iidx_v, rows_v, sem):
    wid = lax.axis_index("s") * NC + lax.axis_index("c")
    base = wid * B_PER_W
    pltpu.sync_copy(uidx_hbm.at[pl.ds(base, B_PER_W)], uidx_v)
    pltpu.sync_copy(iidx_hbm.at[pl.ds(base, B_PER_W)], iidx_v)
    for tab_hbm, idx_v, out_hbm in ((utab_hbm, uidx_v, uout_hbm),
                                    (itab_hbm, iidx_v, iout_hbm)):
        pltpu.async_copy(tab_hbm.at[idx_v], rows_v, sem).wait()
        pltpu.sync_copy(rows_v, out_hbm.at[pl.ds(base, B_PER_W)])


@jax.jit
def _sc_gather(user, item, user_table, item_table):
    mesh = plsc.VectorSubcoreMesh(core_axis_name="c", subcore_axis_name="s")
    k = functools.partial(
        pl.kernel,
        mesh=mesh,
        out_type=[jax.ShapeDtypeStruct((BATCH, EMBED), jnp.float32),
                  jax.ShapeDtypeStruct((BATCH, EMBED), jnp.float32)],
        scratch_types=[
            pltpu.VMEM((B_PER_W,), jnp.int32),
            pltpu.VMEM((B_PER_W,), jnp.int32),
            pltpu.VMEM((B_PER_W, EMBED), jnp.float32),
            pltpu.SemaphoreType.DMA,
        ],
        compiler_params=pltpu.CompilerParams(use_tc_tiling_on_sc=False),
    )(_sc_gather_body)
    return k(user, item, user_table, item_table)


def _mlp_body(u_ref, i_ref, w1_ref, b1_ref, w2_ref, b2_ref, o_ref):
    x = jnp.dot(u_ref[...], w1_ref[0:EMBED, :], preferred_element_type=jnp.float32)
    x = x + jnp.dot(i_ref[...], w1_ref[EMBED:2 * EMBED, :],
                    preferred_element_type=jnp.float32)
    x = jnp.maximum(x + b1_ref[...], 0.0)
    o_ref[...] = jnp.dot(x, w2_ref[...], preferred_element_type=jnp.float32) + b2_ref[...]


@jax.jit
def _tc_mlp(u_emb, i_emb, W1, b1, W2, b2):
    R = 2048
    grid = (BATCH // R,)
    return pl.pallas_call(
        _mlp_body,
        grid=grid,
        in_specs=[
            pl.BlockSpec((R, EMBED), lambda r: (r, 0)),
            pl.BlockSpec((R, EMBED), lambda r: (r, 0)),
            pl.BlockSpec((2 * EMBED, 128), lambda r: (0, 0)),
            pl.BlockSpec((1, 128), lambda r: (0, 0)),
            pl.BlockSpec((128, 5), lambda r: (0, 0)),
            pl.BlockSpec((1, 5), lambda r: (0, 0)),
        ],
        out_specs=pl.BlockSpec((R, 5), lambda r: (r, 0)),
        out_shape=jax.ShapeDtypeStruct((BATCH, 5), jnp.float32),
    )(u_emb, i_emb, W1, b1.reshape(1, 128), W2, b2.reshape(1, 5))


def kernel(user, item, user_table, item_table, W1, b1, W2, b2):
    u_emb, i_emb = _sc_gather(user, item, user_table, item_table)
    return _tc_mlp(u_emb, i_emb, W1, b1, W2, b2)


# speedup vs baseline: 2.7951x; 2.7951x over previous
import functools

import jax
import jax.numpy as jnp
from jax import lax
from jax.experimental import pallas as pl
from jax.experimental.pallas import tpu as pltpu
from jax.experimental.pallas import tpu_sc as plsc

NUM_ROWS = 1000000
BATCH = 16384
EMBED = 32
NC, NS = 2, 16
NW = NC * NS
B_PER_W = BATCH // NW
G = 16
HALF = 8
NGROUPS = B_PER_W // G
LAST_BLK = (NUM_ROWS // 128 - 1) * 128
TAIL0 = (NUM_ROWS // 128) * 128
TAIL_SLOT = HALF * EMBED


def _sc_gather_body(uidx_hbm, iidx_hbm, utab_hbm, itab_hbm,
                    utail_hbm, itail_hbm, uout_hbm, iout_hbm,
                    idx_v, stage_v, ring_v, sem):
    wid = lax.axis_index("s") * NC + lax.axis_index("c")
    base = wid * B_PER_W
    iota = lax.iota(jnp.int32, G)
    for tab_hbm, tail_hbm, sidx_hbm, out_hbm in (
            (utab_hbm, utail_hbm, uidx_hbm, uout_hbm),
            (itab_hbm, itail_hbm, iidx_hbm, iout_hbm)):
        pltpu.sync_copy(sidx_hbm.at[pl.ds(base, B_PER_W)], idx_v)
        pltpu.sync_copy(tail_hbm, ring_v.at[pl.ds(TAIL_SLOT, EMBED)])

        def group(g, carry):
            iv = idx_v[pl.ds(g * G, G)]
            bv = jnp.minimum(jnp.bitwise_and(iv, -128), LAST_BLK)
            tail = iv >= TAIL0
            cv = jnp.where(tail, iv - TAIL0, iv - bv)
            rbv = jnp.where(tail, jnp.full((G,), TAIL_SLOT, jnp.int32),
                            jnp.bitwise_and(iota, HALF - 1) * EMBED)
            for h in range(G // HALF):
                copies = []
                for s in range(HALF):
                    t = h * HALF + s
                    off = pl.multiple_of(bv[t], 128)
                    copies.append(pltpu.async_copy(
                        tab_hbm.at[:, pl.ds(off, 128)],
                        ring_v.at[pl.ds(s * EMBED, EMBED)], sem))
                for c in copies:
                    c.wait()
                for s in range(HALF):
                    t = h * HALF + s
                    rows = rbv[t] + iota
                    csp = jnp.full((G,), cv[t], jnp.int32)
                    lo = plsc.load_gather(ring_v, [rows, csp])
                    hi = plsc.load_gather(ring_v, [rows + G, csp])
                    e = g * G + t
                    stage_v[e, pl.ds(0, G)] = lo
                    stage_v[e, pl.ds(G, G)] = hi
            return carry
        lax.fori_loop(0, NGROUPS, group, None)
        pltpu.sync_copy(stage_v, out_hbm.at[pl.ds(base, B_PER_W)])


@jax.jit
def _sc_gather(user, item, utabT, itabT, utail, itail):
    mesh = plsc.VectorSubcoreMesh(core_axis_name="c", subcore_axis_name="s")
    k = functools.partial(
        pl.kernel,
        mesh=mesh,
        out_type=[jax.ShapeDtypeStruct((BATCH, EMBED), jnp.float32),
                  jax.ShapeDtypeStruct((BATCH, EMBED), jnp.float32)],
        scratch_types=[
            pltpu.VMEM((B_PER_W,), jnp.int32),
            pltpu.VMEM((B_PER_W, EMBED), jnp.float32),
            pltpu.VMEM(((HALF + 1) * EMBED, 128), jnp.float32),
            pltpu.SemaphoreType.DMA,
        ],
        compiler_params=pltpu.CompilerParams(use_tc_tiling_on_sc=True,
                                             needs_layout_passes=False),
    )(_sc_gather_body)
    return k(user, item, utabT, itabT, utail, itail)


def _mlp_body(u_ref, i_ref, w1_ref, b1_ref, w2_ref, b2_ref, o_ref):
    x = jnp.dot(u_ref[...], w1_ref[0:EMBED, :], preferred_element_type=jnp.float32)
    x = x + jnp.dot(i_ref[...], w1_ref[EMBED:2 * EMBED, :],
                    preferred_element_type=jnp.float32)
    x = jnp.maximum(x + b1_ref[...], 0.0)
    o_ref[...] = jnp.dot(x, w2_ref[...], preferred_element_type=jnp.float32) + b2_ref[...]


@jax.jit
def _tc_mlp(u_emb, i_emb, W1, b1, W2, b2):
    R = 2048
    grid = (BATCH // R,)
    return pl.pallas_call(
        _mlp_body,
        grid=grid,
        in_specs=[
            pl.BlockSpec((R, EMBED), lambda r: (r, 0)),
            pl.BlockSpec((R, EMBED), lambda r: (r, 0)),
            pl.BlockSpec((2 * EMBED, 128), lambda r: (0, 0)),
            pl.BlockSpec((1, 128), lambda r: (0, 0)),
            pl.BlockSpec((128, 5), lambda r: (0, 0)),
            pl.BlockSpec((1, 5), lambda r: (0, 0)),
        ],
        out_specs=pl.BlockSpec((R, 5), lambda r: (r, 0)),
        out_shape=jax.ShapeDtypeStruct((BATCH, 5), jnp.float32),
    )(u_emb, i_emb, W1, b1.reshape(1, 128), W2, b2.reshape(1, 5))


def kernel(user, item, user_table, item_table, W1, b1, W2, b2):
    utabT = user_table.T
    itabT = item_table.T
    utail = jnp.pad(utabT[:, TAIL0:], ((0, 0), (0, 128 - (NUM_ROWS - TAIL0))))
    itail = jnp.pad(itabT[:, TAIL0:], ((0, 0), (0, 128 - (NUM_ROWS - TAIL0))))
    u_emb, i_emb = _sc_gather(user, item, utabT, itabT, utail, itail)
    return _tc_mlp(u_emb, i_emb, W1, b1, W2, b2)


# speedup vs baseline: 3.2873x; 1.1761x over previous
import functools

import jax
import jax.numpy as jnp
from jax import lax
from jax.experimental import pallas as pl
from jax.experimental.pallas import tpu as pltpu
from jax.experimental.pallas import tpu_sc as plsc

NUM_ROWS = 1000000
BATCH = 16384
EMBED = 32
NC, NS = 2, 16
NW = NC * NS
B_PER_W = BATCH // NW
H = 8
PHASE = B_PER_W // 2
LAST_BLK = (NUM_ROWS // 128 - 1) * 128
TAIL0 = (NUM_ROWS // 128) * 128
TAIL_SLOT = 2 * H * EMBED


def _sc_gather_body(uidx_hbm, iidx_hbm, utab_hbm, itab_hbm,
                    utail_hbm, itail_hbm, uout_hbm, iout_hbm,
                    idx_v, stage_v, ring_v, sem_a, sem_b):
    wid = lax.axis_index("s") * NC + lax.axis_index("c")
    base = wid * B_PER_W
    iota = lax.iota(jnp.int32, 16)

    def halfvecs(off, poff):
        hv = idx_v[pl.ds(off, 16)]
        bv = jnp.minimum(jnp.bitwise_and(hv, -128), LAST_BLK)
        tail = hv >= TAIL0
        cv = jnp.where(tail, hv - TAIL0, hv - bv)
        rbv = jnp.where(tail, jnp.full((16,), TAIL_SLOT, jnp.int32),
                        poff + iota * EMBED)
        return bv, cv, rbv

    def make_table_loop(tab_hbm):
        def issue(off, poff, sem):
            bv, _, _ = halfvecs(off, poff)
            for s in range(H):
                blk = pl.multiple_of(bv[s], 128)
                pltpu.async_copy(tab_hbm.at[:, pl.ds(blk, 128)],
                                 ring_v.at[pl.ds(poff + s * EMBED, EMBED)],
                                 sem)

        def drain(poff, sem):
            for s in range(H):
                pltpu.make_async_copy(
                    tab_hbm.at[:, pl.ds(0, 128)],
                    ring_v.at[pl.ds(poff + s * EMBED, EMBED)], sem).wait()

        def consume(off, poff, erow):
            _, cv, rbv = halfvecs(off, poff)
            for s in range(H):
                rows = rbv[s] + iota
                csp = jnp.full((16,), cv[s], jnp.int32)
                lo = plsc.load_gather(ring_v, [rows, csp])
                hi = plsc.load_gather(ring_v, [rows + 16, csp])
                stage_v[erow + s, pl.ds(0, 16)] = lo
                stage_v[erow + s, pl.ds(16, 16)] = hi
        return issue, drain, consume

    POFF_A = 0
    POFF_B = H * EMBED
    for tab_hbm, tail_hbm, sidx_hbm, out_hbm in (
            (utab_hbm, utail_hbm, uidx_hbm, uout_hbm),
            (itab_hbm, itail_hbm, iidx_hbm, iout_hbm)):
        issue, drain, consume = make_table_loop(tab_hbm)
        pltpu.sync_copy(sidx_hbm.at[pl.ds(base, B_PER_W)],
                        idx_v.at[pl.ds(0, B_PER_W)])
        pltpu.sync_copy(tail_hbm, ring_v.at[pl.ds(TAIL_SLOT, EMBED)])
        for p in range(2):
            pbase = p * PHASE

            def step(k, carry, pbase=pbase):
                o0 = pbase + k * 2 * H
                issue(o0 + H, POFF_B, sem_b)
                drain(POFF_A, sem_a)
                consume(o0, POFF_A, k * 2 * H)
                issue(o0 + 2 * H, POFF_A, sem_a)
                drain(POFF_B, sem_b)
                consume(o0 + H, POFF_B, k * 2 * H + H)
                return carry

            n_halves = PHASE // H
            issue(pbase, POFF_A, sem_a)
            lax.fori_loop(0, n_halves // 2 - 2, step, None)
            olast = pbase + (n_halves - 4) * H
            issue(olast + H, POFF_B, sem_b)
            drain(POFF_A, sem_a)
            consume(olast, POFF_A, (n_halves - 4) * H)
            issue(olast + 2 * H, POFF_A, sem_a)
            drain(POFF_B, sem_b)
            consume(olast + H, POFF_B, (n_halves - 3) * H)
            issue(olast + 3 * H, POFF_B, sem_b)
            drain(POFF_A, sem_a)
            consume(olast + 2 * H, POFF_A, (n_halves - 2) * H)
            drain(POFF_B, sem_b)
            consume(olast + 3 * H, POFF_B, (n_halves - 1) * H)
            pltpu.sync_copy(stage_v,
                            out_hbm.at[pl.ds(base + pbase, PHASE)])


@jax.jit
def _sc_gather(user, item, utabT, itabT, utail, itail):
    mesh = plsc.VectorSubcoreMesh(core_axis_name="c", subcore_axis_name="s")
    k = functools.partial(
        pl.kernel,
        mesh=mesh,
        out_type=[jax.ShapeDtypeStruct((BATCH, EMBED), jnp.float32),
                  jax.ShapeDtypeStruct((BATCH, EMBED), jnp.float32)],
        scratch_types=[
            pltpu.VMEM((B_PER_W + 16,), jnp.int32),
            pltpu.VMEM((PHASE, EMBED), jnp.float32),
            pltpu.VMEM(((2 * H + 1) * EMBED, 128), jnp.float32),
            pltpu.SemaphoreType.DMA,
            pltpu.SemaphoreType.DMA,
        ],
        compiler_params=pltpu.CompilerParams(use_tc_tiling_on_sc=True,
                                             needs_layout_passes=False),
    )(_sc_gather_body)
    return k(user, item, utabT, itabT, utail, itail)


def _mlp_body(u_ref, i_ref, w1_ref, b1_ref, w2_ref, b2_ref, o_ref):
    x = jnp.dot(u_ref[...], w1_ref[0:EMBED, :], preferred_element_type=jnp.float32)
    x = x + jnp.dot(i_ref[...], w1_ref[EMBED:2 * EMBED, :],
                    preferred_element_type=jnp.float32)
    x = jnp.maximum(x + b1_ref[...], 0.0)
    o_ref[...] = jnp.dot(x, w2_ref[...], preferred_element_type=jnp.float32) + b2_ref[...]


@jax.jit
def _tc_mlp(u_emb, i_emb, W1, b1, W2, b2):
    R = 2048
    grid = (BATCH // R,)
    return pl.pallas_call(
        _mlp_body,
        grid=grid,
        in_specs=[
            pl.BlockSpec((R, EMBED), lambda r: (r, 0)),
            pl.BlockSpec((R, EMBED), lambda r: (r, 0)),
            pl.BlockSpec((2 * EMBED, 128), lambda r: (0, 0)),
            pl.BlockSpec((1, 128), lambda r: (0, 0)),
            pl.BlockSpec((128, 5), lambda r: (0, 0)),
            pl.BlockSpec((1, 5), lambda r: (0, 0)),
        ],
        out_specs=pl.BlockSpec((R, 5), lambda r: (r, 0)),
        out_shape=jax.ShapeDtypeStruct((BATCH, 5), jnp.float32),
    )(u_emb, i_emb, W1, b1.reshape(1, 128), W2, b2.reshape(1, 5))


def kernel(user, item, user_table, item_table, W1, b1, W2, b2):
    utabT = user_table.T
    itabT = item_table.T
    utail = jnp.pad(utabT[:, TAIL0:], ((0, 0), (0, 128 - (NUM_ROWS - TAIL0))))
    itail = jnp.pad(itabT[:, TAIL0:], ((0, 0), (0, 128 - (NUM_ROWS - TAIL0))))
    u_emb, i_emb = _sc_gather(user, item, utabT, itabT, utail, itail)
    return _tc_mlp(u_emb, i_emb, W1, b1, W2, b2)
